# baseline (speedup 1.0000x reference)
import functools

import jax
import jax.numpy as jnp
from jax import lax
from jax.experimental import pallas as pl
from jax.experimental.pallas import tpu as pltpu
from jax.experimental.pallas import tpu_sc as plsc

NU = 25000
NI = 25000
N = NU + NI
NPAD = 50048
PADN = NPAD - 1
D = 32
E = 1600000
NLAYERS = 3
BQ = 16384

NC, NS = 2, 16
NW = NC * NS
CH = 400
NCHUNK = 128
EPT = CH * NCHUNK
E_PAD = EPT * NW
RPW = NPAD // NS
QPW = BQ // NW

_MESH = plsc.VectorSubcoreMesh(core_axis_name="c", subcore_axis_name="s")
_SC_PARAMS = pltpu.CompilerParams(use_tc_tiling_on_sc=False)


def _wid():
    return lax.axis_index("s") * NC + lax.axis_index("c")


@functools.partial(
    pl.kernel,
    out_type=jax.ShapeDtypeStruct((NC * NPAD,), jnp.float32),
    mesh=_MESH,
    compiler_params=_SC_PARAMS,
    scratch_types=[
        pltpu.VMEM_SHARED((NPAD,), jnp.float32),
        pltpu.VMEM((CH,), jnp.int32),
        pltpu.VMEM((CH,), jnp.int32),
        pltpu.VMEM((CH,), jnp.float32),
        pltpu.SemaphoreType.DMA,
        pltpu.SemaphoreType.DMA,
    ],
)
def _k_deg(dst_hbm, ones_hbm, zeros1_hbm, degp_hbm,
           deg_sh, dst0_v, dst1_v, ones_v, isem0, isem1):
    cid = lax.axis_index("c")
    sid = lax.axis_index("s")
    ebase = _wid() * EPT
    pltpu.sync_copy(zeros1_hbm.at[pl.ds(sid * RPW, RPW)],
                    deg_sh.at[pl.ds(sid * RPW, RPW)])
    pltpu.sync_copy(ones_hbm, ones_v)
    plsc.subcore_barrier()

    dst_v = (dst0_v, dst1_v)
    isem = (isem0, isem1)

    def issue(c, k):
        pltpu.async_copy(dst_hbm.at[pl.ds(ebase + c * CH, CH)],
                         dst_v[k], isem[k])

    def wait(k):
        pltpu.make_async_copy(dst_hbm.at[pl.ds(ebase, CH)],
                              dst_v[k], isem[k]).wait()

    issue(0, 0)
    issue(1, 1)

    def body(p, _):
        for k in range(2):
            c = 2 * p + k
            wait(k)
            pltpu.sync_copy(ones_v, deg_sh.at[dst_v[k]], add=True)

            @pl.when(c + 2 < NCHUNK)
            def _():
                issue(c + 2, k)
        return 0

    lax.fori_loop(0, NCHUNK // 2, body, 0)
    plsc.subcore_barrier()
    pltpu.sync_copy(deg_sh.at[pl.ds(sid * RPW, RPW)],
                    degp_hbm.at[pl.ds(cid * NPAD + sid * RPW, RPW)])


@functools.partial(
    pl.kernel,
    out_type=jax.ShapeDtypeStruct((NC, NPAD, D), jnp.float32),
    mesh=_MESH,
    compiler_params=_SC_PARAMS,
    scratch_types=[
        pltpu.VMEM_SHARED((NPAD, D), jnp.float32),
        pltpu.VMEM((2, CH), jnp.int32),
        pltpu.VMEM((2, CH), jnp.int32),
        pltpu.VMEM((CH, D), jnp.float32),
        pltpu.VMEM((CH, D), jnp.float32),
    ] + [pltpu.SemaphoreType.DMA] * 4,
)
def _k_layer(y_hbm, ei_hbm, zeros2_hbm, part_hbm, acc_sh,
             idx0, idx1, rows0, rows1, is0, is1, gs0, gs1):
    cid = lax.axis_index("c")
    sid = lax.axis_index("s")
    ebase = _wid() * EPT
    pltpu.sync_copy(zeros2_hbm.at[pl.ds(sid * RPW, RPW)],
                    acc_sh.at[pl.ds(sid * RPW, RPW)])
    plsc.subcore_barrier()

    idx_v = (idx0, idx1)
    isem = (is0, is1)
    rows_v = (rows0, rows1)
    gsem = (gs0, gs1)

    def issue_idx(c, k):
        pltpu.async_copy(ei_hbm.at[:, pl.ds(ebase + c * CH, CH)],
                         idx_v[k], isem[k])

    def wait_idx(k):
        pltpu.make_async_copy(ei_hbm.at[:, pl.ds(ebase, CH)],
                              idx_v[k], isem[k]).wait()

    def issue_gather(k, r):
        pltpu.async_copy(y_hbm.at[idx_v[k].at[0]], rows_v[r], gsem[r])

    def wait_gather(k, r):
        pltpu.make_async_copy(y_hbm.at[idx_v[k].at[0]], rows_v[r],
                              gsem[r]).wait()

    def step(i, s, last_idx, last_gather):
        s1 = 1 - s
        wait_gather(s, s)
        pltpu.sync_copy(rows_v[s], acc_sh.at[idx_v[s].at[1]], add=True)
        if not last_idx:
            issue_idx(i + 2, s)
        if not last_gather:
            wait_idx(s1)
            issue_gather(s1, s1)

    issue_idx(0, 0)
    issue_idx(1, 1)
    wait_idx(0)
    issue_gather(0, 0)

    def body(p, _):
        i = p * 2
        step(i + 0, 0, last_idx=False, last_gather=False)
        step(i + 1, 1, last_idx=False, last_gather=False)
        return 0

    lax.fori_loop(0, 63, body, 0)
    step(126, 0, last_idx=True, last_gather=False)
    step(127, 1, last_idx=True, last_gather=True)

    plsc.subcore_barrier()
    pltpu.sync_copy(acc_sh.at[pl.ds(sid * RPW, RPW)],
                    part_hbm.at[cid, pl.ds(sid * RPW, RPW)])


@functools.partial(
    pl.kernel,
    out_type=[
        jax.ShapeDtypeStruct((BQ, D), jnp.float32),
        jax.ShapeDtypeStruct((BQ, D), jnp.float32),
    ],
    mesh=_MESH,
    compiler_params=_SC_PARAMS,
    scratch_types=[
        pltpu.VMEM((QPW,), jnp.int32),
        pltpu.VMEM((QPW,), jnp.int32),
        pltpu.VMEM((QPW, D), jnp.float32),
        pltpu.VMEM((QPW, D), jnp.float32),
        pltpu.SemaphoreType.DMA,
    ],
)
def _k_gather(s_hbm, user_hbm, pos_hbm, ug_hbm, pg_hbm,
              iu_v, ip_v, urows_v, prows_v, sem):
    wid = _wid()
    base = wid * QPW
    pltpu.sync_copy(user_hbm.at[pl.ds(base, QPW)], iu_v)
    pltpu.sync_copy(pos_hbm.at[pl.ds(base, QPW)], ip_v)

    def shift(j, _):
        off = pl.multiple_of(j * 16, 8)
        ip_v[pl.ds(off, 16)] = ip_v[pl.ds(off, 16)] + NU
        return 0

    lax.fori_loop(0, QPW // 16, shift, 0)

    pltpu.async_copy(s_hbm.at[iu_v], urows_v, sem).wait()
    pltpu.async_copy(s_hbm.at[ip_v], prows_v, sem).wait()
    pltpu.sync_copy(urows_v, ug_hbm.at[pl.ds(base, QPW)])
    pltpu.sync_copy(prows_v, pg_hbm.at[pl.ds(base, QPW)])


_ROWS = 3128
_GRID = NPAD // _ROWS


def _k2_body(x0_ref, d0_ref, d1_ref, dinv_ref, y0_ref):
    deg = d0_ref[0] + d1_ref[0]
    dinv = jnp.where(deg > 0, lax.rsqrt(jnp.maximum(deg, 1.0)), 0.0)
    dinv_ref[...] = dinv
    y0_ref[...] = x0_ref[...] * dinv


def _k4_body(a0_ref, a1_ref, dinv_ref, s_ref, s_out_ref, y_ref):
    dinv = dinv_ref[...]
    x = (a0_ref[0] + a1_ref[0]) * dinv
    s_out_ref[...] = s_ref[...] + x
    y_ref[...] = x * dinv


def _k4_last_body(a0_ref, a1_ref, dinv_ref, s_ref, s_out_ref):
    x = (a0_ref[0] + a1_ref[0]) * dinv_ref[...]
    s_out_ref[...] = s_ref[...] + x


def _k_dot_body(u_ref, p_ref, o_ref):
    o_ref[...] = jnp.sum(u_ref[...] * p_ref[...], axis=1, keepdims=True) * (
        1.0 / 16.0)


_bs_rows = pl.BlockSpec((_ROWS, D), lambda i: (i, 0))
_bs_col = pl.BlockSpec((_ROWS, 1), lambda i: (i, 0))
_bs_deg0 = pl.BlockSpec((1, _ROWS, 1), lambda i: (0, i, 0))
_bs_deg1 = pl.BlockSpec((1, _ROWS, 1), lambda i: (1, i, 0))
_bs_part0 = pl.BlockSpec((1, _ROWS, D), lambda i: (0, i, 0))
_bs_part1 = pl.BlockSpec((1, _ROWS, D), lambda i: (1, i, 0))

_k2 = pl.pallas_call(
    _k2_body,
    grid=(_GRID,),
    in_specs=[_bs_rows, _bs_deg0, _bs_deg1],
    out_specs=[_bs_col, _bs_rows],
    out_shape=[
        jax.ShapeDtypeStruct((NPAD, 1), jnp.float32),
        jax.ShapeDtypeStruct((NPAD, D), jnp.float32),
    ],
)

_k4 = pl.pallas_call(
    _k4_body,
    grid=(_GRID,),
    in_specs=[_bs_part0, _bs_part1, _bs_col, _bs_rows],
    out_specs=[_bs_rows, _bs_rows],
    out_shape=[
        jax.ShapeDtypeStruct((NPAD, D), jnp.float32),
        jax.ShapeDtypeStruct((NPAD, D), jnp.float32),
    ],
)

_k4_last = pl.pallas_call(
    _k4_last_body,
    grid=(_GRID,),
    in_specs=[_bs_part0, _bs_part1, _bs_col, _bs_rows],
    out_specs=_bs_rows,
    out_shape=jax.ShapeDtypeStruct((NPAD, D), jnp.float32),
)

_QROWS = 2048
_qs_rows = pl.BlockSpec((_QROWS, D), lambda i: (i, 0))
_qs_col = pl.BlockSpec((_QROWS, 1), lambda i: (i, 0))

_k_dot_tc = pl.pallas_call(
    _k_dot_body,
    grid=(BQ // _QROWS,),
    in_specs=[_qs_rows, _qs_rows],
    out_specs=_qs_col,
    out_shape=jax.ShapeDtypeStruct((BQ, 1), jnp.float32),
)


def kernel(Gu, Gi, edge_index, user, pos):
    npe = E_PAD - E
    seq = jnp.arange(npe, dtype=jnp.int32)
    ei = jnp.concatenate(
        [edge_index,
         jnp.stack([jnp.full((npe,), PADN, jnp.int32), seq % N])], axis=1)
    dst_deg = jnp.concatenate([edge_index[1], N + seq % (NPAD - N)])
    x0 = jnp.concatenate(
        [Gu, Gi, jnp.zeros((NPAD - N, D), jnp.float32)], axis=0)
    ones = jnp.ones((CH,), jnp.float32)
    zeros1 = jnp.zeros((NPAD,), jnp.float32)
    zeros2 = jnp.zeros((NPAD, D), jnp.float32)

    degp = _k_deg(dst_deg, ones, zeros1).reshape(NC, NPAD, 1)
    dinv, y = _k2(x0, degp, degp)

    s = x0
    for layer in range(NLAYERS):
        part = _k_layer(y, ei, zeros2)
        if layer < NLAYERS - 1:
            s, y = _k4(part, part, dinv, s)
        else:
            s = _k4_last(part, part, dinv, s)

    ug, pg = _k_gather(s, user, pos)
    return _k_dot_tc(ug, pg).reshape(BQ)


# speedup vs baseline: 1.0505x; 1.0505x over previous
import functools

import jax
import jax.numpy as jnp
from jax import lax
from jax.experimental import pallas as pl
from jax.experimental.pallas import tpu as pltpu
from jax.experimental.pallas import tpu_sc as plsc

NU = 25000
NI = 25000
N = NU + NI
NPAD = 50048
PADN = NPAD - 1
D = 32
E = 1600000
NLAYERS = 3
BQ = 16384

NC, NS = 2, 16
NW = NC * NS
CH = 400
NCHUNK = 128
EPT = CH * NCHUNK
E_PAD = EPT * NW
RPW = NPAD // NS
QPW = BQ // NW

_MESH = plsc.VectorSubcoreMesh(core_axis_name="c", subcore_axis_name="s")
_SC_PARAMS = pltpu.CompilerParams(use_tc_tiling_on_sc=False)


def _wid():
    return lax.axis_index("s") * NC + lax.axis_index("c")


@functools.partial(
    pl.kernel,
    out_type=jax.ShapeDtypeStruct((NC * NPAD,), jnp.float32),
    mesh=_MESH,
    compiler_params=_SC_PARAMS,
    scratch_types=[
        pltpu.VMEM_SHARED((NPAD,), jnp.float32),
        pltpu.VMEM((CH,), jnp.int32),
        pltpu.VMEM((CH,), jnp.int32),
        pltpu.VMEM((CH,), jnp.float32),
        pltpu.SemaphoreType.DMA,
        pltpu.SemaphoreType.DMA,
    ],
)
def _k_deg(dst_hbm, ones_hbm, zeros1_hbm, degp_hbm,
           deg_sh, dst0_v, dst1_v, ones_v, isem0, isem1):
    cid = lax.axis_index("c")
    sid = lax.axis_index("s")
    ebase = _wid() * EPT
    pltpu.sync_copy(zeros1_hbm.at[pl.ds(sid * RPW, RPW)],
                    deg_sh.at[pl.ds(sid * RPW, RPW)])
    pltpu.sync_copy(ones_hbm, ones_v)
    plsc.subcore_barrier()

    dst_v = (dst0_v, dst1_v)
    isem = (isem0, isem1)

    def issue(c, k):
        pltpu.async_copy(dst_hbm.at[pl.ds(ebase + c * CH, CH)],
                         dst_v[k], isem[k])

    def wait(k):
        pltpu.make_async_copy(dst_hbm.at[pl.ds(ebase, CH)],
                              dst_v[k], isem[k]).wait()

    issue(0, 0)
    issue(1, 1)

    def body(p, _):
        for k in range(2):
            c = 2 * p + k
            wait(k)
            pltpu.sync_copy(ones_v, deg_sh.at[dst_v[k]], add=True)

            @pl.when(c + 2 < NCHUNK)
            def _():
                issue(c + 2, k)
        return 0

    lax.fori_loop(0, NCHUNK // 2, body, 0)
    plsc.subcore_barrier()
    pltpu.sync_copy(deg_sh.at[pl.ds(sid * RPW, RPW)],
                    degp_hbm.at[pl.ds(cid * NPAD + sid * RPW, RPW)])


@functools.partial(
    pl.kernel,
    out_type=jax.ShapeDtypeStruct((NC, NPAD, D), jnp.float32),
    mesh=_MESH,
    compiler_params=_SC_PARAMS,
    scratch_types=[
        pltpu.VMEM_SHARED((NPAD, D), jnp.float32),
        pltpu.VMEM((CH,), jnp.int32),
        pltpu.VMEM((CH,), jnp.int32),
        pltpu.VMEM((CH,), jnp.int32),
        pltpu.VMEM((CH,), jnp.int32),
        pltpu.VMEM((CH, D), jnp.float32),
        pltpu.VMEM((CH, D), jnp.float32),
    ] + [pltpu.SemaphoreType.DMA] * 4,
)
def _k_layer(src_hbm, dst_hbm, y_hbm, zeros2_hbm, part_hbm, acc_sh,
             src0, src1, dst0, dst1, rows0, rows1, is0, is1, gs0, gs1):
    cid = lax.axis_index("c")
    sid = lax.axis_index("s")
    ebase = _wid() * EPT
    pltpu.sync_copy(zeros2_hbm.at[pl.ds(sid * RPW, RPW)],
                    acc_sh.at[pl.ds(sid * RPW, RPW)])
    plsc.subcore_barrier()

    src_v = (src0, src1)
    dst_v = (dst0, dst1)
    isem = (is0, is1)
    rows_v = (rows0, rows1)
    gsem = (gs0, gs1)

    def issue_idx(c, k):
        pltpu.async_copy(src_hbm.at[pl.ds(ebase + c * CH, CH)],
                         src_v[k], isem[k])
        pltpu.async_copy(dst_hbm.at[pl.ds(ebase + c * CH, CH)],
                         dst_v[k], isem[k])

    def wait_idx(k):
        pltpu.make_async_copy(src_hbm.at[pl.ds(ebase, CH)],
                              src_v[k], isem[k]).wait()
        pltpu.make_async_copy(dst_hbm.at[pl.ds(ebase, CH)],
                              dst_v[k], isem[k]).wait()

    def issue_gather(k, r):
        pltpu.async_copy(y_hbm.at[src_v[k]], rows_v[r], gsem[r])

    def wait_gather(k, r):
        pltpu.make_async_copy(y_hbm.at[src_v[k]], rows_v[r],
                              gsem[r]).wait()

    def step(i, s, last_idx, last_gather):
        s1 = 1 - s
        wait_gather(s, s)
        pltpu.sync_copy(rows_v[s], acc_sh.at[dst_v[s]], add=True)
        if not last_idx:
            issue_idx(i + 2, s)
        if not last_gather:
            wait_idx(s1)
            issue_gather(s1, s1)

    issue_idx(0, 0)
    issue_idx(1, 1)
    wait_idx(0)
    issue_gather(0, 0)

    def body(p, _):
        i = p * 2
        step(i + 0, 0, last_idx=False, last_gather=False)
        step(i + 1, 1, last_idx=False, last_gather=False)
        return 0

    lax.fori_loop(0, 63, body, 0)
    step(126, 0, last_idx=True, last_gather=False)
    step(127, 1, last_idx=True, last_gather=True)

    plsc.subcore_barrier()
    pltpu.sync_copy(acc_sh.at[pl.ds(sid * RPW, RPW)],
                    part_hbm.at[cid, pl.ds(sid * RPW, RPW)])


@functools.partial(
    pl.kernel,
    out_type=[
        jax.ShapeDtypeStruct((BQ, D), jnp.float32),
        jax.ShapeDtypeStruct((BQ, D), jnp.float32),
    ],
    mesh=_MESH,
    compiler_params=_SC_PARAMS,
    scratch_types=[
        pltpu.VMEM((QPW,), jnp.int32),
        pltpu.VMEM((QPW,), jnp.int32),
        pltpu.VMEM((QPW, D), jnp.float32),
        pltpu.VMEM((QPW, D), jnp.float32),
        pltpu.SemaphoreType.DMA,
    ],
)
def _k_gather(s_hbm, user_hbm, pos_hbm, ug_hbm, pg_hbm,
              iu_v, ip_v, urows_v, prows_v, sem):
    wid = _wid()
    base = wid * QPW
    pltpu.sync_copy(user_hbm.at[pl.ds(base, QPW)], iu_v)
    pltpu.sync_copy(pos_hbm.at[pl.ds(base, QPW)], ip_v)

    def shift(j, _):
        off = pl.multiple_of(j * 16, 8)
        ip_v[pl.ds(off, 16)] = ip_v[pl.ds(off, 16)] + NU
        return 0

    lax.fori_loop(0, QPW // 16, shift, 0)

    pltpu.async_copy(s_hbm.at[iu_v], urows_v, sem).wait()
    pltpu.async_copy(s_hbm.at[ip_v], prows_v, sem).wait()
    pltpu.sync_copy(urows_v, ug_hbm.at[pl.ds(base, QPW)])
    pltpu.sync_copy(prows_v, pg_hbm.at[pl.ds(base, QPW)])


_ROWS = 3128
_GRID = NPAD // _ROWS


def _k2_body(x0_ref, d0_ref, d1_ref, dinv_ref, y0_ref):
    deg = d0_ref[0] + d1_ref[0]
    dinv = jnp.where(deg > 0, lax.rsqrt(jnp.maximum(deg, 1.0)), 0.0)
    dinv_ref[...] = dinv
    y0_ref[...] = x0_ref[...] * dinv


def _k4_body(a0_ref, a1_ref, dinv_ref, s_ref, s_out_ref, y_ref):
    dinv = dinv_ref[...]
    x = (a0_ref[0] + a1_ref[0]) * dinv
    s_out_ref[...] = s_ref[...] + x
    y_ref[...] = x * dinv


def _k4_last_body(a0_ref, a1_ref, dinv_ref, s_ref, s_out_ref):
    x = (a0_ref[0] + a1_ref[0]) * dinv_ref[...]
    s_out_ref[...] = s_ref[...] + x


def _k_dot_body(u_ref, p_ref, o_ref):
    o_ref[...] = jnp.sum(u_ref[...] * p_ref[...], axis=1, keepdims=True) * (
        1.0 / 16.0)


_bs_rows = pl.BlockSpec((_ROWS, D), lambda i: (i, 0))
_bs_col = pl.BlockSpec((_ROWS, 1), lambda i: (i, 0))
_bs_deg0 = pl.BlockSpec((1, _ROWS, 1), lambda i: (0, i, 0))
_bs_deg1 = pl.BlockSpec((1, _ROWS, 1), lambda i: (1, i, 0))
_bs_part0 = pl.BlockSpec((1, _ROWS, D), lambda i: (0, i, 0))
_bs_part1 = pl.BlockSpec((1, _ROWS, D), lambda i: (1, i, 0))

_k2 = pl.pallas_call(
    _k2_body,
    grid=(_GRID,),
    in_specs=[_bs_rows, _bs_deg0, _bs_deg1],
    out_specs=[_bs_col, _bs_rows],
    out_shape=[
        jax.ShapeDtypeStruct((NPAD, 1), jnp.float32),
        jax.ShapeDtypeStruct((NPAD, D), jnp.float32),
    ],
)

_k4 = pl.pallas_call(
    _k4_body,
    grid=(_GRID,),
    in_specs=[_bs_part0, _bs_part1, _bs_col, _bs_rows],
    out_specs=[_bs_rows, _bs_rows],
    out_shape=[
        jax.ShapeDtypeStruct((NPAD, D), jnp.float32),
        jax.ShapeDtypeStruct((NPAD, D), jnp.float32),
    ],
)

_k4_last = pl.pallas_call(
    _k4_last_body,
    grid=(_GRID,),
    in_specs=[_bs_part0, _bs_part1, _bs_col, _bs_rows],
    out_specs=_bs_rows,
    out_shape=jax.ShapeDtypeStruct((NPAD, D), jnp.float32),
)

_QROWS = 2048
_qs_rows = pl.BlockSpec((_QROWS, D), lambda i: (i, 0))
_qs_col = pl.BlockSpec((_QROWS, 1), lambda i: (i, 0))

_k_dot_tc = pl.pallas_call(
    _k_dot_body,
    grid=(BQ // _QROWS,),
    in_specs=[_qs_rows, _qs_rows],
    out_specs=_qs_col,
    out_shape=jax.ShapeDtypeStruct((BQ, 1), jnp.float32),
)


def kernel(Gu, Gi, edge_index, user, pos):
    npe = E_PAD - E
    seq = jnp.arange(npe, dtype=jnp.int32)
    src_p = jnp.concatenate([edge_index[0], jnp.full((npe,), PADN,
                                                     jnp.int32)])
    dst_p = jnp.concatenate([edge_index[1], seq % N])
    dst_deg = jnp.concatenate([edge_index[1], N + seq % (NPAD - N)])
    x0 = jnp.concatenate(
        [Gu, Gi, jnp.zeros((NPAD - N, D), jnp.float32)], axis=0)
    ones = jnp.ones((CH,), jnp.float32)
    zeros1 = jnp.zeros((NPAD,), jnp.float32)
    zeros2 = jnp.zeros((NPAD, D), jnp.float32)

    degp = _k_deg(dst_deg, ones, zeros1).reshape(NC, NPAD, 1)
    dinv, y = _k2(x0, degp, degp)

    s = x0
    for layer in range(NLAYERS):
        part = _k_layer(src_p, dst_p, y, zeros2)
        if layer < NLAYERS - 1:
            s, y = _k4(part, part, dinv, s)
        else:
            s = _k4_last(part, part, dinv, s)

    ug, pg = _k_gather(s, user, pos)
    return _k_dot_tc(ug, pg).reshape(BQ)


# speedup vs baseline: 2.1250x; 2.0228x over previous
import functools

import jax
import jax.numpy as jnp
from jax import lax
from jax.experimental import pallas as pl
from jax.experimental.pallas import tpu as pltpu
from jax.experimental.pallas import tpu_sc as plsc

NU = 25000
NI = 25000
N = NU + NI
NPAD = 50048
PADN = NPAD - 1
D = 32
E = 1600000
NLAYERS = 3
BQ = 16384

NC, NS = 2, 16
NW = NC * NS
CH = 400
NCHUNK = 128
EPT = CH * NCHUNK
E_PAD = EPT * NW
RPW = NPAD // NS
QPW = BQ // NW

_MESH = plsc.VectorSubcoreMesh(core_axis_name="c", subcore_axis_name="s")
_SC_PARAMS = pltpu.CompilerParams(use_tc_tiling_on_sc=False)


def _wid():
    return lax.axis_index("s") * NC + lax.axis_index("c")


@functools.partial(
    pl.kernel,
    out_type=jax.ShapeDtypeStruct((NC * NPAD,), jnp.float32),
    mesh=_MESH,
    compiler_params=_SC_PARAMS,
    scratch_types=[
        pltpu.VMEM_SHARED((NPAD,), jnp.float32),
        pltpu.VMEM((CH,), jnp.int32),
        pltpu.VMEM((CH,), jnp.int32),
        pltpu.VMEM((CH,), jnp.float32),
        pltpu.SemaphoreType.DMA,
        pltpu.SemaphoreType.DMA,
    ],
)
def _k_deg(dst_hbm, ones_hbm, zeros1_hbm, degp_hbm,
           deg_sh, dst0_v, dst1_v, ones_v, isem0, isem1):
    cid = lax.axis_index("c")
    sid = lax.axis_index("s")
    ebase = _wid() * EPT
    pltpu.sync_copy(zeros1_hbm.at[pl.ds(sid * RPW, RPW)],
                    deg_sh.at[pl.ds(sid * RPW, RPW)])
    pltpu.sync_copy(ones_hbm, ones_v)
    plsc.subcore_barrier()

    dst_v = (dst0_v, dst1_v)
    isem = (isem0, isem1)

    def issue(c, k):
        pltpu.async_copy(dst_hbm.at[pl.ds(ebase + c * CH, CH)],
                         dst_v[k], isem[k])

    def wait(k):
        pltpu.make_async_copy(dst_hbm.at[pl.ds(ebase, CH)],
                              dst_v[k], isem[k]).wait()

    issue(0, 0)
    issue(1, 1)

    def body(p, _):
        for k in range(2):
            c = 2 * p + k
            wait(k)
            pltpu.sync_copy(ones_v, deg_sh.at[dst_v[k]], add=True)

            @pl.when(c + 2 < NCHUNK)
            def _():
                issue(c + 2, k)
        return 0

    lax.fori_loop(0, NCHUNK // 2, body, 0)
    plsc.subcore_barrier()
    pltpu.sync_copy(deg_sh.at[pl.ds(sid * RPW, RPW)],
                    degp_hbm.at[pl.ds(cid * NPAD + sid * RPW, RPW)])


@functools.partial(
    pl.kernel,
    out_type=jax.ShapeDtypeStruct((NC, NPAD, D), jnp.float32),
    mesh=_MESH,
    compiler_params=_SC_PARAMS,
    scratch_types=[
        pltpu.VMEM_SHARED((NPAD, D), jnp.float32),
        pltpu.VMEM((CH,), jnp.int32),
        pltpu.VMEM((CH,), jnp.int32),
        pltpu.VMEM((CH,), jnp.int32),
        pltpu.VMEM((CH,), jnp.int32),
        pltpu.VMEM((CH, D), jnp.float32),
        pltpu.VMEM((CH, D), jnp.float32),
    ] + [pltpu.SemaphoreType.DMA] * 4,
)
def _k_layer(src_hbm, dst_hbm, y_hbm, zeros2_hbm, part_hbm, acc_sh,
             src0, src1, dst0, dst1, rows0, rows1, is0, is1, gs0, gs1):
    cid = lax.axis_index("c")
    sid = lax.axis_index("s")
    ebase = _wid() * EPT
    pltpu.sync_copy(zeros2_hbm.at[pl.ds(sid * RPW, RPW)],
                    acc_sh.at[pl.ds(sid * RPW, RPW)])
    plsc.subcore_barrier()

    src_v = (src0, src1)
    dst_v = (dst0, dst1)
    isem = (is0, is1)
    rows_v = (rows0, rows1)
    gsem = (gs0, gs1)

    def issue_idx(c, k):
        pltpu.async_copy(src_hbm.at[pl.ds(ebase + c * CH, CH)],
                         src_v[k], isem[k])
        pltpu.async_copy(dst_hbm.at[pl.ds(ebase + c * CH, CH)],
                         dst_v[k], isem[k])

    def wait_idx(k):
        pltpu.make_async_copy(src_hbm.at[pl.ds(ebase, CH)],
                              src_v[k], isem[k]).wait()
        pltpu.make_async_copy(dst_hbm.at[pl.ds(ebase, CH)],
                              dst_v[k], isem[k]).wait()

    def issue_gather(k, r):
        pltpu.async_copy(y_hbm.at[src_v[k]], rows_v[r], gsem[r])

    def wait_gather(k, r):
        pltpu.make_async_copy(y_hbm.at[src_v[k]], rows_v[r],
                              gsem[r]).wait()

    def step(i, s, last_idx, last_gather):
        s1 = 1 - s
        wait_gather(s, s)
        pltpu.sync_copy(rows_v[s], acc_sh.at[dst_v[s]], add=True)
        if not last_idx:
            issue_idx(i + 2, s)
        if not last_gather:
            wait_idx(s1)
            issue_gather(s1, s1)

    issue_idx(0, 0)
    issue_idx(1, 1)
    wait_idx(0)
    issue_gather(0, 0)

    def body(p, _):
        i = p * 2
        step(i + 0, 0, last_idx=False, last_gather=False)
        step(i + 1, 1, last_idx=False, last_gather=False)
        return 0

    lax.fori_loop(0, 63, body, 0)
    step(126, 0, last_idx=True, last_gather=False)
    step(127, 1, last_idx=True, last_gather=True)

    plsc.subcore_barrier()
    pltpu.sync_copy(acc_sh.at[pl.ds(sid * RPW, RPW)],
                    part_hbm.at[cid, pl.ds(sid * RPW, RPW)])


@functools.partial(
    pl.kernel,
    out_type=[
        jax.ShapeDtypeStruct((BQ, D), jnp.float32),
        jax.ShapeDtypeStruct((BQ, D), jnp.float32),
    ],
    mesh=_MESH,
    compiler_params=_SC_PARAMS,
    scratch_types=[
        pltpu.VMEM((QPW,), jnp.int32),
        pltpu.VMEM((QPW,), jnp.int32),
        pltpu.VMEM((QPW, D), jnp.float32),
        pltpu.VMEM((QPW, D), jnp.float32),
        pltpu.SemaphoreType.DMA,
    ],
)
def _k_gather(s_hbm, user_hbm, pos_hbm, ug_hbm, pg_hbm,
              iu_v, ip_v, urows_v, prows_v, sem):
    wid = _wid()
    base = wid * QPW
    pltpu.sync_copy(user_hbm.at[pl.ds(base, QPW)], iu_v)
    pltpu.sync_copy(pos_hbm.at[pl.ds(base, QPW)], ip_v)

    def shift(j, _):
        off = pl.multiple_of(j * 16, 8)
        ip_v[pl.ds(off, 16)] = ip_v[pl.ds(off, 16)] + NU
        return 0

    lax.fori_loop(0, QPW // 16, shift, 0)

    pltpu.async_copy(s_hbm.at[iu_v], urows_v, sem).wait()
    pltpu.async_copy(s_hbm.at[ip_v], prows_v, sem).wait()
    pltpu.sync_copy(urows_v, ug_hbm.at[pl.ds(base, QPW)])
    pltpu.sync_copy(prows_v, pg_hbm.at[pl.ds(base, QPW)])


_ROWS = 3128
_GRID = NPAD // _ROWS


def _k2_body(x0_ref, d0_ref, d1_ref, dinv_ref, y0_ref):
    deg = d0_ref[0] + d1_ref[0]
    dinv = jnp.where(deg > 0, lax.rsqrt(jnp.maximum(deg, 1.0)), 0.0)
    dinv_ref[...] = dinv
    y0_ref[...] = x0_ref[...] * dinv


def _k4_body(a0_ref, a1_ref, dinv_ref, s_ref, s_out_ref, y_ref):
    dinv = dinv_ref[...]
    x = (a0_ref[0] + a1_ref[0]) * dinv
    s_out_ref[...] = s_ref[...] + x
    y_ref[...] = x * dinv


def _k4_last_body(a0_ref, a1_ref, dinv_ref, s_ref, s_out_ref):
    x = (a0_ref[0] + a1_ref[0]) * dinv_ref[...]
    s_out_ref[...] = s_ref[...] + x


def _k_dot_body(u_ref, p_ref, o_ref):
    o_ref[...] = jnp.sum(u_ref[...] * p_ref[...], axis=1, keepdims=True) * (
        1.0 / 16.0)


_bs_rows = pl.BlockSpec((_ROWS, D), lambda i: (i, 0))
_bs_col = pl.BlockSpec((_ROWS, 1), lambda i: (i, 0))
_bs_deg0 = pl.BlockSpec((1, _ROWS, 1), lambda i: (0, i, 0))
_bs_deg1 = pl.BlockSpec((1, _ROWS, 1), lambda i: (1, i, 0))
_bs_part0 = pl.BlockSpec((1, _ROWS, D), lambda i: (0, i, 0))
_bs_part1 = pl.BlockSpec((1, _ROWS, D), lambda i: (1, i, 0))

_k2 = pl.pallas_call(
    _k2_body,
    grid=(_GRID,),
    in_specs=[_bs_rows, _bs_deg0, _bs_deg1],
    out_specs=[_bs_col, _bs_rows],
    out_shape=[
        jax.ShapeDtypeStruct((NPAD, 1), jnp.float32),
        jax.ShapeDtypeStruct((NPAD, D), jnp.float32),
    ],
)

_k4 = pl.pallas_call(
    _k4_body,
    grid=(_GRID,),
    in_specs=[_bs_part0, _bs_part1, _bs_col, _bs_rows],
    out_specs=[_bs_rows, _bs_rows],
    out_shape=[
        jax.ShapeDtypeStruct((NPAD, D), jnp.float32),
        jax.ShapeDtypeStruct((NPAD, D), jnp.float32),
    ],
)

_k4_last = pl.pallas_call(
    _k4_last_body,
    grid=(_GRID,),
    in_specs=[_bs_part0, _bs_part1, _bs_col, _bs_rows],
    out_specs=_bs_rows,
    out_shape=jax.ShapeDtypeStruct((NPAD, D), jnp.float32),
)

_QROWS = 2048
_qs_rows = pl.BlockSpec((_QROWS, D), lambda i: (i, 0))
_qs_col = pl.BlockSpec((_QROWS, 1), lambda i: (i, 0))

_k_dot_tc = pl.pallas_call(
    _k_dot_body,
    grid=(BQ // _QROWS,),
    in_specs=[_qs_rows, _qs_rows],
    out_specs=_qs_col,
    out_shape=jax.ShapeDtypeStruct((BQ, 1), jnp.float32),
)


def kernel(Gu, Gi, edge_index, user, pos):
    npe = E_PAD - E
    seq = jnp.arange(npe, dtype=jnp.int32)
    src_p = jnp.concatenate([edge_index[0], N + seq % (NPAD - N)])
    dst_p = jnp.concatenate([edge_index[1], seq % N])
    dst_deg = jnp.concatenate([edge_index[1], N + seq % (NPAD - N)])
    x0 = jnp.concatenate(
        [Gu, Gi, jnp.zeros((NPAD - N, D), jnp.float32)], axis=0)
    ones = jnp.ones((CH,), jnp.float32)
    zeros1 = jnp.zeros((NPAD,), jnp.float32)
    zeros2 = jnp.zeros((NPAD, D), jnp.float32)

    degp = _k_deg(dst_deg, ones, zeros1).reshape(NC, NPAD, 1)
    dinv, y = _k2(x0, degp, degp)

    s = x0
    for layer in range(NLAYERS):
        part = _k_layer(src_p, dst_p, y, zeros2)
        if layer < NLAYERS - 1:
            s, y = _k4(part, part, dinv, s)
        else:
            s = _k4_last(part, part, dinv, s)

    ug, pg = _k_gather(s, user, pos)
    return _k_dot_tc(ug, pg).reshape(BQ)
